# baseline (speedup 1.0000x reference)
import functools

import jax
import jax.numpy as jnp
from jax.experimental import pallas as pl

_HIDDEN = 128
_OUT = 2
_EPS = 1e-5


def _mlp_block_kernel(x_ref, ln1w_ref, ln1b_ref, w1t_ref, b1_ref,
                      ln2w_ref, ln2b_ref, w2t_ref, b2_ref, out_ref):
    x = x_ref[...]
    mu = jnp.mean(x, axis=-1, keepdims=True)
    xc = x - mu
    var = jnp.mean(xc * xc, axis=-1, keepdims=True)
    xn = xc * jax.lax.rsqrt(var + _EPS)
    xn = xn * ln1w_ref[...] + ln1b_ref[...]

    h = jnp.dot(xn, w1t_ref[...], preferred_element_type=jnp.float32)
    h = jax.nn.sigmoid(h + b1_ref[...])

    mu2 = jnp.mean(h, axis=-1, keepdims=True)
    hc = h - mu2
    var2 = jnp.mean(hc * hc, axis=-1, keepdims=True)
    hn = hc * jax.lax.rsqrt(var2 + _EPS)
    hn = hn * ln2w_ref[...] + ln2b_ref[...]

    out = jnp.dot(hn, w2t_ref[...], preferred_element_type=jnp.float32)
    out_ref[...] = out + b2_ref[...]


@functools.partial(jax.jit, static_argnames=("block_rows",))
def _run(causal, ln1_w, ln1_b, W1, b1, ln2_w, ln2_b, W2, b2, block_rows=10000):
    n_rows = causal.shape[0]
    grid = (n_rows // block_rows,)

    row2 = lambda s: pl.BlockSpec(s, lambda i: (0, 0))
    out = pl.pallas_call(
        _mlp_block_kernel,
        grid=grid,
        in_specs=[
            pl.BlockSpec((block_rows, _HIDDEN), lambda i: (i, 0)),
            row2((1, _HIDDEN)),
            row2((1, _HIDDEN)),
            row2((_HIDDEN, _HIDDEN)),
            row2((1, _HIDDEN)),
            row2((1, _HIDDEN)),
            row2((1, _HIDDEN)),
            row2((_HIDDEN, _OUT)),
            row2((1, _OUT)),
        ],
        out_specs=pl.BlockSpec((block_rows, _OUT), lambda i: (i, 0)),
        out_shape=jax.ShapeDtypeStruct((n_rows, _OUT), jnp.float32),
    )(
        causal,
        ln1_w.reshape(1, _HIDDEN),
        ln1_b.reshape(1, _HIDDEN),
        W1.T,
        b1.reshape(1, _HIDDEN),
        ln2_w.reshape(1, _HIDDEN),
        ln2_b.reshape(1, _HIDDEN),
        W2.T,
        b2.reshape(1, _OUT),
    )
    return out


def kernel(causal, ln1_w, ln1_b, W1, b1, ln2_w, ln2_b, W2, b2):
    return _run(causal, ln1_w, ln1_b, W1, b1, ln2_w, ln2_b, W2, b2)


# speedup vs baseline: 1.0122x; 1.0122x over previous
import functools

import jax
import jax.numpy as jnp
from jax.experimental import pallas as pl

_HIDDEN = 128
_OUT = 2
_EPS = 1e-5


def _mlp_block_kernel(x_ref, ln1w_ref, ln1b_ref, w1t_ref, b1_ref,
                      ln2w_ref, ln2b_ref, w2t_ref, b2_ref, out_ref):
    x = x_ref[...]
    mu = jnp.mean(x, axis=-1, keepdims=True)
    xc = x - mu
    var = jnp.mean(xc * xc, axis=-1, keepdims=True)
    xn = xc * jax.lax.rsqrt(var + _EPS)
    xn = xn * ln1w_ref[...] + ln1b_ref[...]

    h = jnp.dot(xn, w1t_ref[...], preferred_element_type=jnp.float32)
    h = jax.nn.sigmoid(h + b1_ref[...])

    mu2 = jnp.mean(h, axis=-1, keepdims=True)
    hc = h - mu2
    var2 = jnp.mean(hc * hc, axis=-1, keepdims=True)
    hn = hc * jax.lax.rsqrt(var2 + _EPS)
    hn = hn * ln2w_ref[...] + ln2b_ref[...]

    out = jnp.dot(hn, w2t_ref[...], preferred_element_type=jnp.float32)
    out_ref[...] = out + b2_ref[...]


@functools.partial(jax.jit, static_argnames=("block_rows",))
def _run(causal, ln1_w, ln1_b, W1, b1, ln2_w, ln2_b, W2, b2, block_rows=4000):
    n_rows = causal.shape[0]
    grid = (n_rows // block_rows,)

    row2 = lambda s: pl.BlockSpec(s, lambda i: (0, 0))
    out = pl.pallas_call(
        _mlp_block_kernel,
        grid=grid,
        in_specs=[
            pl.BlockSpec((block_rows, _HIDDEN), lambda i: (i, 0)),
            row2((1, _HIDDEN)),
            row2((1, _HIDDEN)),
            row2((_HIDDEN, _HIDDEN)),
            row2((1, _HIDDEN)),
            row2((1, _HIDDEN)),
            row2((1, _HIDDEN)),
            row2((_HIDDEN, _OUT)),
            row2((1, _OUT)),
        ],
        out_specs=pl.BlockSpec((block_rows, _OUT), lambda i: (i, 0)),
        out_shape=jax.ShapeDtypeStruct((n_rows, _OUT), jnp.float32),
    )(
        causal,
        ln1_w.reshape(1, _HIDDEN),
        ln1_b.reshape(1, _HIDDEN),
        W1.T,
        b1.reshape(1, _HIDDEN),
        ln2_w.reshape(1, _HIDDEN),
        ln2_b.reshape(1, _HIDDEN),
        W2.T,
        b2.reshape(1, _OUT),
    )
    return out


def kernel(causal, ln1_w, ln1_b, W1, b1, ln2_w, ln2_b, W2, b2):
    return _run(causal, ln1_w, ln1_b, W1, b1, ln2_w, ln2_b, W2, b2)
